# baseline (speedup 1.0000x reference)
import functools

import jax
import jax.numpy as jnp
from jax.experimental import pallas as pl
from jax.experimental.pallas import tpu as pltpu

_BLK = 128
_CHUNK = 256
_IOU_THR = 0.5
_MAX_DET = 100
_NCLS = 80


def _nms_mask_kernel(ms_ref, b_ref, bT_ref, sup_ref, s_blk):
    npad = b_ref.shape[0]
    nblk = npad // _BLK

    sup_ref[...] = jnp.zeros((1, npad), jnp.int32)

    lane_b = jax.lax.broadcasted_iota(jnp.int32, (1, _BLK), 1)
    lane_c = jax.lax.broadcasted_iota(jnp.int32, (1, _CHUNK), 1)

    maxseg = ms_ref[0]
    nch_reach = (maxseg + _BLK - 2 + _CHUNK - 1) // _CHUNK

    def blk_body(i, carry):
        start = i * _BLK
        blk = b_ref[pl.ds(start, _BLK), :]
        x1b = blk[:, 0:1]
        y1b = blk[:, 1:2]
        x2b = blk[:, 2:3]
        y2b = blk[:, 3:4]
        area_b = (x2b - x1b) * (y2b - y1b)

        bt = bT_ref[:, pl.ds(start, _BLK)]
        x1r = bt[0:1, :]
        y1r = bt[1:2, :]
        x2r = bt[2:3, :]
        y2r = bt[3:4, :]
        area_r = (x2r - x1r) * (y2r - y1r)
        wb = jnp.maximum(jnp.minimum(x2b, x2r) - jnp.maximum(x1b, x1r), 0.0)
        hb = jnp.maximum(jnp.minimum(y2b, y2r) - jnp.maximum(y1b, y1r), 0.0)
        interb = wb * hb
        ioub = interb / (area_b + area_r - interb + 1e-9)
        overb = ioub > _IOU_THR

        supb0 = sup_ref[:, pl.ds(start, _BLK)]
        row_i = jax.lax.broadcasted_iota(jnp.int32, (_BLK, _BLK), 0)
        col_i = jax.lax.broadcasted_iota(jnp.int32, (_BLK, _BLK), 1)
        act0 = jnp.any(overb & (row_i > col_i), axis=0, keepdims=True)
        act0 = (act0 & (supb0 == 0)).astype(jnp.int32)
        any_act = jnp.max(act0) > 0

        @pl.when(any_act)
        def _store_blk():
            s_blk[...] = overb.astype(jnp.int32)

        def cond(c):
            _, a = c
            return jnp.max(a) > 0

        def body(c):
            sb, a = c
            j = jnp.min(jnp.where(a > 0, lane_b, _BLK))
            row = s_blk[pl.ds(j, 1), :]
            sb2 = sb | (((lane_b > j) & (row > 0)).astype(jnp.int32))
            a2 = a & (1 - sb2) & ((lane_b != j).astype(jnp.int32))
            return sb2, a2

        supb, _ = jax.lax.while_loop(cond, body, (supb0, act0))
        sup_ref[:, pl.ds(start, _BLK)] = supb

        kept = (supb == 0).astype(jnp.float32)
        nch_rem = (npad - start - _BLK + _CHUNK - 1) // _CHUNK
        nch = jnp.minimum(nch_reach, nch_rem)

        def col_body(m, c):
            cs = jnp.minimum(start + _BLK + m * _CHUNK, npad - _CHUNK)
            bt_c = bT_ref[:, pl.ds(cs, _CHUNK)]
            x1c = bt_c[0:1, :]
            y1c = bt_c[1:2, :]
            x2c = bt_c[2:3, :]
            y2c = bt_c[3:4, :]
            area_c = (x2c - x1c) * (y2c - y1c)
            wc = jnp.maximum(jnp.minimum(x2b, x2c) - jnp.maximum(x1b, x1c), 0.0)
            hc = jnp.maximum(jnp.minimum(y2b, y2c) - jnp.maximum(y1b, y1c), 0.0)
            ic = wc * hc
            iouc = ic / (area_b + area_c - ic + 1e-9)
            overc = (iouc > _IOU_THR).astype(jnp.float32)
            contrib = jnp.dot(kept, overc, preferred_element_type=jnp.float32)
            valid = (cs + lane_c) >= (start + _BLK)
            cur = sup_ref[:, pl.ds(cs, _CHUNK)]
            sup_ref[:, pl.ds(cs, _CHUNK)] = cur | (
                (contrib > 0.0) & valid
            ).astype(jnp.int32)
            return c

        jax.lax.fori_loop(0, nch, col_body, 0)
        return carry

    jax.lax.fori_loop(0, nblk, blk_body, 0)


@functools.partial(jax.jit, static_argnames=())
def kernel(boxes, scores, class_idxs):
    n = boxes.shape[0]
    npad = ((n + _BLK - 1) // _BLK) * _BLK

    max_coord = jnp.max(boxes) + 1.0
    cls_f = class_idxs.astype(boxes.dtype)
    neg_s, sx1, sy1, sx2, sy2, s_cls = jax.lax.sort(
        (-scores, boxes[:, 0], boxes[:, 1], boxes[:, 2], boxes[:, 3], cls_f),
        num_keys=1,
        is_stable=True,
    )
    s_scores = -neg_s

    rank = jax.lax.iota(jnp.int32, n)
    c2, cx1, cy1, cx2, cy2, r2 = jax.lax.sort(
        (s_cls, sx1, sy1, sx2, sy2, rank),
        num_keys=1,
        is_stable=True,
    )

    off = c2 * max_coord
    b_sorted = jnp.stack([cx1 + off, cy1 + off, cx2 + off, cy2 + off], axis=1)
    b_pad = jnp.zeros((npad, 4), jnp.float32).at[:n, :].set(b_sorted)
    bT_pad = b_pad.T

    counts = jnp.sum(
        c2[None, :] == jnp.arange(_NCLS, dtype=c2.dtype)[:, None], axis=1
    )
    maxseg = jnp.max(counts).astype(jnp.int32)

    sup = pl.pallas_call(
        _nms_mask_kernel,
        out_shape=jax.ShapeDtypeStruct((1, npad), jnp.int32),
        in_specs=[
            pl.BlockSpec(memory_space=pltpu.SMEM),
            pl.BlockSpec(memory_space=pltpu.VMEM),
            pl.BlockSpec(memory_space=pltpu.VMEM),
        ],
        scratch_shapes=[pltpu.VMEM((_BLK, _BLK), jnp.int32)],
    )(maxseg[None], b_pad, bT_pad)

    sup_rank = jnp.zeros((n,), jnp.int32).at[r2].set(sup[0, :n])
    kept_scores = jnp.where(sup_rank > 0, -jnp.inf, s_scores)
    _, topk_idx = jax.lax.top_k(kept_scores, _MAX_DET)
    out_boxes = jnp.stack([sx1, sy1, sx2, sy2], axis=1)[topk_idx]
    return (
        out_boxes,
        s_scores[topk_idx],
        s_cls[topk_idx].astype(class_idxs.dtype),
    )


# speedup vs baseline: 1.3698x; 1.3698x over previous
import functools

import jax
import jax.numpy as jnp
from jax.experimental import pallas as pl
from jax.experimental.pallas import tpu as pltpu

_BLK = 128
_CHUNK = 2048
_IOU_THR = 0.5
_MAX_DET = 100


def _nms_mask_kernel(b_ref, bT_ref, sup_ref, s_blk):
    npad = b_ref.shape[0]
    nblk = npad // _BLK

    sup_ref[...] = jnp.zeros((1, npad), jnp.int32)

    lane_b = jax.lax.broadcasted_iota(jnp.int32, (1, _BLK), 1)

    def blk_body(i, carry):
        start = i * _BLK
        blk = b_ref[pl.ds(start, _BLK), :]
        x1b = blk[:, 0:1]
        y1b = blk[:, 1:2]
        x2b = blk[:, 2:3]
        y2b = blk[:, 3:4]
        area_b = (x2b - x1b) * (y2b - y1b)

        bt = bT_ref[:, pl.ds(start, _BLK)]
        x1r = bt[0:1, :]
        y1r = bt[1:2, :]
        x2r = bt[2:3, :]
        y2r = bt[3:4, :]
        area_r = (x2r - x1r) * (y2r - y1r)
        wb = jnp.maximum(jnp.minimum(x2b, x2r) - jnp.maximum(x1b, x1r), 0.0)
        hb = jnp.maximum(jnp.minimum(y2b, y2r) - jnp.maximum(y1b, y1r), 0.0)
        interb = wb * hb
        ioub = interb / (area_b + area_r - interb + 1e-9)
        overb = ioub > _IOU_THR
        s_blk[...] = overb.astype(jnp.int32)

        supb0 = sup_ref[:, pl.ds(start, _BLK)]
        row_i = jax.lax.broadcasted_iota(jnp.int32, (_BLK, _BLK), 0)
        col_i = jax.lax.broadcasted_iota(jnp.int32, (_BLK, _BLK), 1)
        act0 = jnp.any(overb & (row_i > col_i), axis=0, keepdims=True)
        act0 = (act0 & (supb0 == 0)).astype(jnp.int32)

        def cond(c):
            _, a = c
            return jnp.max(a) > 0

        def body(c):
            sb, a = c
            j = jnp.min(jnp.where(a > 0, lane_b, _BLK))
            row = s_blk[pl.ds(j, 1), :]
            sb2 = sb | (((lane_b > j) & (row > 0)).astype(jnp.int32))
            a2 = a & (1 - sb2) & ((lane_b != j).astype(jnp.int32))
            return sb2, a2

        supb, _ = jax.lax.while_loop(cond, body, (supb0, act0))
        sup_ref[:, pl.ds(start, _BLK)] = supb

        kept = (supb == 0).astype(jnp.float32)
        lane_c = jax.lax.broadcasted_iota(jnp.int32, (1, _CHUNK), 1)
        nch = (npad - start - _BLK + _CHUNK - 1) // _CHUNK

        def col_body(m, c):
            cs = jnp.minimum(start + _BLK + m * _CHUNK, npad - _CHUNK)
            bt_c = bT_ref[:, pl.ds(cs, _CHUNK)]
            x1c = bt_c[0:1, :]
            y1c = bt_c[1:2, :]
            x2c = bt_c[2:3, :]
            y2c = bt_c[3:4, :]
            area_c = (x2c - x1c) * (y2c - y1c)
            wc = jnp.maximum(jnp.minimum(x2b, x2c) - jnp.maximum(x1b, x1c), 0.0)
            hc = jnp.maximum(jnp.minimum(y2b, y2c) - jnp.maximum(y1b, y1c), 0.0)
            ic = wc * hc
            iouc = ic / (area_b + area_c - ic + 1e-9)
            overc = (iouc > _IOU_THR).astype(jnp.float32)
            contrib = jnp.dot(kept, overc, preferred_element_type=jnp.float32)
            valid = (cs + lane_c) >= (start + _BLK)
            cur = sup_ref[:, pl.ds(cs, _CHUNK)]
            sup_ref[:, pl.ds(cs, _CHUNK)] = cur | (
                (contrib > 0.0) & valid
            ).astype(jnp.int32)
            return c

        jax.lax.fori_loop(0, nch, col_body, 0)
        return carry

    jax.lax.fori_loop(0, nblk, blk_body, 0)


@functools.partial(jax.jit, static_argnames=())
def kernel(boxes, scores, class_idxs):
    n = boxes.shape[0]
    npad = ((n + _BLK - 1) // _BLK) * _BLK

    max_coord = jnp.max(boxes) + 1.0
    cls_f = class_idxs.astype(boxes.dtype)
    neg_s, sx1, sy1, sx2, sy2, s_cls = jax.lax.sort(
        (-scores, boxes[:, 0], boxes[:, 1], boxes[:, 2], boxes[:, 3], cls_f),
        num_keys=1,
        is_stable=True,
    )
    s_scores = -neg_s
    off = s_cls * max_coord
    b_sorted = jnp.stack([sx1 + off, sy1 + off, sx2 + off, sy2 + off], axis=1)
    b_pad = jnp.zeros((npad, 4), jnp.float32).at[:n, :].set(b_sorted)
    bT_pad = b_pad.T

    sup = pl.pallas_call(
        _nms_mask_kernel,
        out_shape=jax.ShapeDtypeStruct((1, npad), jnp.int32),
        scratch_shapes=[pltpu.VMEM((_BLK, _BLK), jnp.int32)],
    )(b_pad, bT_pad)

    suppressed = sup[0, :n] > 0
    kept_scores = jnp.where(suppressed, -jnp.inf, s_scores)
    _, topk_idx = jax.lax.top_k(kept_scores, _MAX_DET)
    out_boxes = jnp.stack([sx1, sy1, sx2, sy2], axis=1)[topk_idx]
    return (
        out_boxes,
        s_scores[topk_idx],
        s_cls[topk_idx].astype(class_idxs.dtype),
    )


# speedup vs baseline: 1.4325x; 1.0458x over previous
import functools

import jax
import jax.numpy as jnp
from jax.experimental import pallas as pl
from jax.experimental.pallas import tpu as pltpu

_BLK = 128
_CHUNK = 2048
_IOU_THR = 0.5
_MAX_DET = 100


def _nms_kernel(b_ref, bT_ref, data_ref, out_ref, s_blk, s_sup):
    npad = b_ref.shape[0]
    nblk = npad // _BLK

    s_sup[...] = jnp.zeros((1, npad), jnp.int32)

    lane_b = jax.lax.broadcasted_iota(jnp.int32, (1, _BLK), 1)
    lane_c = jax.lax.broadcasted_iota(jnp.int32, (1, _CHUNK), 1)

    def blk_body(i, carry):
        start = i * _BLK
        blk = b_ref[pl.ds(start, _BLK), :]
        x1b = blk[:, 0:1]
        y1b = blk[:, 1:2]
        x2b = blk[:, 2:3]
        y2b = blk[:, 3:4]
        area_b = (x2b - x1b) * (y2b - y1b)

        bt = bT_ref[:, pl.ds(start, _BLK)]
        x1r = bt[0:1, :]
        y1r = bt[1:2, :]
        x2r = bt[2:3, :]
        y2r = bt[3:4, :]
        area_r = (x2r - x1r) * (y2r - y1r)
        wb = jnp.maximum(jnp.minimum(x2b, x2r) - jnp.maximum(x1b, x1r), 0.0)
        hb = jnp.maximum(jnp.minimum(y2b, y2r) - jnp.maximum(y1b, y1r), 0.0)
        interb = wb * hb
        ioub = interb / (area_b + area_r - interb + 1e-9)
        overb = ioub > _IOU_THR
        s_blk[...] = overb.astype(jnp.int32)

        supb0 = s_sup[:, pl.ds(start, _BLK)]
        row_i = jax.lax.broadcasted_iota(jnp.int32, (_BLK, _BLK), 0)
        col_i = jax.lax.broadcasted_iota(jnp.int32, (_BLK, _BLK), 1)
        act0 = jnp.any(overb & (row_i > col_i), axis=0, keepdims=True)
        act0 = (act0 & (supb0 == 0)).astype(jnp.int32)

        def cond(c):
            _, a = c
            return jnp.max(a) > 0

        def body(c):
            sb, a = c
            j = jnp.min(jnp.where(a > 0, lane_b, _BLK))
            row = s_blk[pl.ds(j, 1), :]
            sb2 = sb | (((lane_b > j) & (row > 0)).astype(jnp.int32))
            a2 = a & (1 - sb2) & ((lane_b != j).astype(jnp.int32))
            return sb2, a2

        supb, _ = jax.lax.while_loop(cond, body, (supb0, act0))
        s_sup[:, pl.ds(start, _BLK)] = supb

        kept = (supb == 0).astype(jnp.float32)
        nch = (npad - start - _BLK + _CHUNK - 1) // _CHUNK

        def col_body(m, c):
            cs = jnp.minimum(start + _BLK + m * _CHUNK, npad - _CHUNK)
            bt_c = bT_ref[:, pl.ds(cs, _CHUNK)]
            x1c = bt_c[0:1, :]
            y1c = bt_c[1:2, :]
            x2c = bt_c[2:3, :]
            y2c = bt_c[3:4, :]
            area_c = (x2c - x1c) * (y2c - y1c)
            wc = jnp.maximum(jnp.minimum(x2b, x2c) - jnp.maximum(x1b, x1c), 0.0)
            hc = jnp.maximum(jnp.minimum(y2b, y2c) - jnp.maximum(y1b, y1c), 0.0)
            ic = wc * hc
            iouc = ic / (area_b + area_c - ic + 1e-9)
            overc = (iouc > _IOU_THR).astype(jnp.float32)
            contrib = jnp.dot(kept, overc, preferred_element_type=jnp.float32)
            valid = (cs + lane_c) >= (start + _BLK)
            cur = s_sup[:, pl.ds(cs, _CHUNK)]
            s_sup[:, pl.ds(cs, _CHUNK)] = cur | (
                (contrib > 0.0) & valid
            ).astype(jnp.int32)
            return c

        jax.lax.fori_loop(0, nch, col_body, 0)
        return carry

    jax.lax.fori_loop(0, nblk, blk_body, 0)

    sup = s_sup[...]
    realrow = data_ref[6:7, :]
    validm = realrow > 0.0
    keptf = ((sup == 0) & validm).astype(jnp.float32)
    suppf = ((sup > 0) & validm).astype(jnp.float32)

    kept2d = jnp.concatenate(
        [keptf[:, r * _BLK:(r + 1) * _BLK] for r in range(nblk)], axis=0
    )
    supp2d = jnp.concatenate(
        [suppf[:, r * _BLK:(r + 1) * _BLK] for r in range(nblk)], axis=0
    )

    row128 = jax.lax.broadcasted_iota(jnp.int32, (_BLK, _BLK), 0)
    col128 = jax.lax.broadcasted_iota(jnp.int32, (_BLK, _BLK), 1)
    lt_incl = (row128 <= col128).astype(jnp.float32)
    rowr = jax.lax.broadcasted_iota(jnp.int32, (nblk, nblk), 0)
    colr = jax.lax.broadcasted_iota(jnp.int32, (nblk, nblk), 1)
    strict_r = (colr < rowr).astype(jnp.float32)

    def ranks(m2d):
        pref = jnp.dot(m2d, lt_incl, preferred_element_type=jnp.float32)
        rowtot = pref[:, _BLK - 1:_BLK]
        excl = jnp.dot(strict_r, rowtot, preferred_element_type=jnp.float32)
        return pref + excl - m2d

    pos_k = ranks(kept2d)
    pos_s = ranks(supp2d)
    total_k = jnp.sum(kept2d)
    slot = jnp.where(kept2d > 0.0, pos_k, total_k + pos_s)
    slot = jnp.where((kept2d > 0.0) | (supp2d > 0.0), slot, 3.0e7)

    sub128 = jax.lax.broadcasted_iota(jnp.int32, (_BLK, _BLK), 0).astype(
        jnp.float32
    )
    acc = jnp.zeros((8, _BLK), jnp.float32)
    for r in range(nblk):
        slot_r = slot[r:r + 1, :]
        q_r = (slot_r == sub128).astype(jnp.float32)
        d_r = data_ref[:, pl.ds(r * _BLK, _BLK)]
        acc = acc + jax.lax.dot_general(
            d_r, q_r, (((1,), (1,)), ((), ())),
            precision=jax.lax.Precision.HIGHEST,
            preferred_element_type=jnp.float32,
        )
    out_ref[...] = acc


@functools.partial(jax.jit, static_argnames=())
def kernel(boxes, scores, class_idxs):
    n = boxes.shape[0]
    npad = ((n + _BLK - 1) // _BLK) * _BLK

    max_coord = jnp.max(boxes) + 1.0
    cls_f = class_idxs.astype(boxes.dtype)
    neg_s, sx1, sy1, sx2, sy2, s_cls = jax.lax.sort(
        (-scores, boxes[:, 0], boxes[:, 1], boxes[:, 2], boxes[:, 3], cls_f),
        num_keys=1,
        is_stable=True,
    )
    s_scores = -neg_s
    off = s_cls * max_coord
    b_sorted = jnp.stack([sx1 + off, sy1 + off, sx2 + off, sy2 + off], axis=1)
    b_pad = jnp.zeros((npad, 4), jnp.float32).at[:n, :].set(b_sorted)
    bT_pad = b_pad.T
    ones_n = jnp.ones((n,), jnp.float32)
    data = jnp.zeros((8, npad), jnp.float32)
    data = data.at[:7, :n].set(
        jnp.stack([sx1, sy1, sx2, sy2, s_scores, s_cls, ones_n])
    )

    out = pl.pallas_call(
        _nms_kernel,
        out_shape=jax.ShapeDtypeStruct((8, _BLK), jnp.float32),
        scratch_shapes=[
            pltpu.VMEM((_BLK, _BLK), jnp.int32),
            pltpu.VMEM((1, npad), jnp.int32),
        ],
    )(b_pad, bT_pad, data)

    out_boxes = out[0:4, :_MAX_DET].T
    return (
        out_boxes,
        out[4, :_MAX_DET],
        out[5, :_MAX_DET].astype(class_idxs.dtype),
    )
